# baseline (speedup 1.0000x reference)
import jax
import jax.numpy as jnp
from jax import lax
from jax.experimental import pallas as pl
from jax.experimental.pallas import tpu as pltpu
from jax.experimental.pallas import tpu_sc as plsc

N_NODES = 10000
N_EDGES = 320000
D = 128
NC = 2
NS = 16
NW = NC * NS
EPW = N_EDGES // NW
C = 64
NFULL = 156
TAIL = EPW - NFULL * C
ROWS_PER_TILE = 624


def _lane_shuffle(x, perm):
    dnums = lax.GatherDimensionNumbers(
        offset_dims=(), collapsed_slice_dims=(0,), start_index_map=(0,))
    return lax.gather(x, perm[:, None], dnums, slice_sizes=(1,),
                      mode=lax.GatherScatterMode.PROMISE_IN_BOUNDS)


def _body(src_hbm, dst_hbm, h_hbm, out_hbm,
          h_sh, si0, di0, si1, di1, rs0, rd0, rs1, rd1, ob,
          sem_s0, sem_d0, sem_s1, sem_d1):
    c = lax.axis_index("c")
    s = lax.axis_index("s")
    wid = s * NC + c

    r0 = s * ROWS_PER_TILE
    pltpu.sync_copy(h_hbm.at[pl.ds(r0, ROWS_PER_TILE)],
                    h_sh.at[pl.ds(r0, ROWS_PER_TILE)])
    tail_rows = NS * ROWS_PER_TILE
    @pl.when(s == 0)
    def _():
        pltpu.sync_copy(h_hbm.at[pl.ds(tail_rows, N_NODES - tail_rows)],
                        h_sh.at[pl.ds(tail_rows, N_NODES - tail_rows)])
    plsc.subcore_barrier()

    lane = lax.iota(jnp.int32, 16)
    perms = [lane ^ (1 << k) for k in range(4)]
    masks = [(lane & (1 << k)) == 0 for k in range(4)]
    base = wid * EPW

    def fetch(t, si, di, rs, rd, ss, sd):
        off = jnp.minimum(base + t * C, N_EDGES - C)
        pltpu.sync_copy(src_hbm.at[pl.ds(off, C)], si)
        pltpu.sync_copy(dst_hbm.at[pl.ds(off, C)], di)
        cs = pltpu.make_async_copy(h_sh.at[si], rs, ss)
        cd = pltpu.make_async_copy(h_sh.at[di], rd, sd)
        cs.start()
        cd.start()

    def dots16(rs, rd, e0):
        def edge_acc(e):
            acc = None
            for j in range(D // 16):
                p = rs[e, pl.ds(j * 16, 16)] * rd[e, pl.ds(j * 16, 16)]
                acc = p if acc is None else acc + p
            return acc

        vec = jnp.zeros((16,), jnp.float32)
        for m in range(8):
            a = edge_acc(e0 + 2 * m)
            b = edge_acc(e0 + 2 * m + 1)
            cv = (jnp.where(masks[0], a, b) +
                  _lane_shuffle(jnp.where(masks[0], b, a), perms[0]))
            for k in (1, 2, 3):
                cv = cv + _lane_shuffle(cv, perms[k])
            vec = jnp.where((lane >> 1) == m, cv, vec)
        return vec

    def compute(t, si, di, rs, rd, ss, sd):
        pltpu.make_async_copy(h_sh.at[si], rs, ss).wait()
        pltpu.make_async_copy(h_sh.at[di], rd, sd).wait()

        def group(g, _):
            ob[pl.ds(g * 16, 16)] = dots16(rs, rd, g * 16)
            return 0

        lax.fori_loop(0, C // 16, group, 0)
        pltpu.sync_copy(ob, out_hbm.at[pl.ds(base + t * C, C)])

    fetch(0, si0, di0, rs0, rd0, sem_s0, sem_d0)

    def pair(u, _):
        t0 = 2 * u
        fetch(t0 + 1, si1, di1, rs1, rd1, sem_s1, sem_d1)
        compute(t0, si0, di0, rs0, rd0, sem_s0, sem_d0)
        fetch(t0 + 2, si0, di0, rs0, rd0, sem_s0, sem_d0)
        compute(t0 + 1, si1, di1, rs1, rd1, sem_s1, sem_d1)
        return 0

    lax.fori_loop(0, NFULL // 2, pair, 0)

    pltpu.make_async_copy(h_sh.at[si0], rs0, sem_s0).wait()
    pltpu.make_async_copy(h_sh.at[di0], rd0, sem_d0).wait()

    toff = base + NFULL * C
    pltpu.sync_copy(src_hbm.at[pl.ds(toff, TAIL)], si0.at[pl.ds(0, TAIL)])
    pltpu.sync_copy(dst_hbm.at[pl.ds(toff, TAIL)], di0.at[pl.ds(0, TAIL)])
    ct_s = pltpu.make_async_copy(h_sh.at[si0.at[pl.ds(0, TAIL)]],
                                 rs0.at[pl.ds(0, TAIL)], sem_s0)
    ct_d = pltpu.make_async_copy(h_sh.at[di0.at[pl.ds(0, TAIL)]],
                                 rd0.at[pl.ds(0, TAIL)], sem_d0)
    ct_s.start()
    ct_d.start()
    ct_s.wait()
    ct_d.wait()
    ob[pl.ds(0, TAIL)] = dots16(rs0, rd0, 0)
    pltpu.sync_copy(ob.at[pl.ds(0, TAIL)], out_hbm.at[pl.ds(toff, TAIL)])


@jax.jit
def _scores(src, dst, h):
    mesh = plsc.VectorSubcoreMesh(core_axis_name="c", subcore_axis_name="s")
    return pl.kernel(
        _body,
        out_type=jax.ShapeDtypeStruct((N_EDGES,), jnp.float32),
        mesh=mesh,
        scratch_types=[
            pltpu.VMEM_SHARED((N_NODES, D), jnp.float32),
            pltpu.VMEM((C,), jnp.int32),
            pltpu.VMEM((C,), jnp.int32),
            pltpu.VMEM((C,), jnp.int32),
            pltpu.VMEM((C,), jnp.int32),
            pltpu.VMEM((C, D), jnp.float32),
            pltpu.VMEM((C, D), jnp.float32),
            pltpu.VMEM((C, D), jnp.float32),
            pltpu.VMEM((C, D), jnp.float32),
            pltpu.VMEM((C,), jnp.float32),
            pltpu.SemaphoreType.DMA,
            pltpu.SemaphoreType.DMA,
            pltpu.SemaphoreType.DMA,
            pltpu.SemaphoreType.DMA,
        ],
    )(src, dst, h)


def kernel(edge_index, h):
    ei = edge_index.astype(jnp.int32)
    scores = _scores(ei[0], ei[1], h)
    return scores.reshape(N_EDGES, 1)


# speedup vs baseline: 1.1523x; 1.1523x over previous
import jax
import jax.numpy as jnp
from jax import lax
from jax.experimental import pallas as pl
from jax.experimental.pallas import tpu as pltpu
from jax.experimental.pallas import tpu_sc as plsc

N_NODES = 10000
N_EDGES = 320000
D = 128
NC = 2
NS = 16
NW = NC * NS
EPW = N_EDGES // NW
C = 64
NFULL = 156
TAIL = EPW - NFULL * C
ROWS_PER_TILE = 624


def _lane_shuffle(x, perm):
    dnums = lax.GatherDimensionNumbers(
        offset_dims=(), collapsed_slice_dims=(0,), start_index_map=(0,))
    return lax.gather(x, perm[:, None], dnums, slice_sizes=(1,),
                      mode=lax.GatherScatterMode.PROMISE_IN_BOUNDS)


def _body(src_hbm, dst_hbm, h_hbm, out_hbm,
          h_sh, si0, di0, si1, di1, rs0, rd0, rs1, rd1, ob,
          sem_s0, sem_d0, sem_s1, sem_d1):
    c = lax.axis_index("c")
    s = lax.axis_index("s")
    wid = s * NC + c

    r0 = s * ROWS_PER_TILE
    pltpu.sync_copy(h_hbm.at[pl.ds(r0, ROWS_PER_TILE)],
                    h_sh.at[pl.ds(r0, ROWS_PER_TILE)])
    tail_rows = NS * ROWS_PER_TILE
    @pl.when(s == 0)
    def _():
        pltpu.sync_copy(h_hbm.at[pl.ds(tail_rows, N_NODES - tail_rows)],
                        h_sh.at[pl.ds(tail_rows, N_NODES - tail_rows)])
    plsc.subcore_barrier()

    lane = lax.iota(jnp.int32, 16)
    perms = [lane ^ sh for sh in (1, 2, 4, 8)]
    base = wid * EPW

    def fetch(t, si, di, rs, rd, ss, sd):
        off = jnp.minimum(base + t * C, N_EDGES - C)
        pltpu.sync_copy(src_hbm.at[pl.ds(off, C)], si)
        pltpu.sync_copy(dst_hbm.at[pl.ds(off, C)], di)
        cs = pltpu.make_async_copy(h_sh.at[si], rs, ss)
        cd = pltpu.make_async_copy(h_sh.at[di], rd, sd)
        cs.start()
        cd.start()

    def dots16(rs, rd, e0):
        vec = jnp.zeros((16,), jnp.float32)
        for i in range(16):
            e = e0 + i
            acc = None
            for j in range(D // 16):
                p = rs[e, pl.ds(j * 16, 16)] * rd[e, pl.ds(j * 16, 16)]
                acc = p if acc is None else acc + p
            for pm in perms:
                acc = acc + _lane_shuffle(acc, pm)
            vec = jnp.where(lane == i, acc, vec)
        return vec

    def compute(t, si, di, rs, rd, ss, sd):
        pltpu.make_async_copy(h_sh.at[si], rs, ss).wait()
        pltpu.make_async_copy(h_sh.at[di], rd, sd).wait()

        def group(g, _):
            ob[pl.ds(g * 16, 16)] = dots16(rs, rd, g * 16)
            return 0

        lax.fori_loop(0, C // 16, group, 0)
        pltpu.sync_copy(ob, out_hbm.at[pl.ds(base + t * C, C)])

    fetch(0, si0, di0, rs0, rd0, sem_s0, sem_d0)

    def pair(u, _):
        t0 = 2 * u
        fetch(t0 + 1, si1, di1, rs1, rd1, sem_s1, sem_d1)
        compute(t0, si0, di0, rs0, rd0, sem_s0, sem_d0)
        fetch(t0 + 2, si0, di0, rs0, rd0, sem_s0, sem_d0)
        compute(t0 + 1, si1, di1, rs1, rd1, sem_s1, sem_d1)
        return 0

    lax.fori_loop(0, NFULL // 2, pair, 0)

    pltpu.make_async_copy(h_sh.at[si0], rs0, sem_s0).wait()
    pltpu.make_async_copy(h_sh.at[di0], rd0, sem_d0).wait()

    toff = base + NFULL * C
    pltpu.sync_copy(src_hbm.at[pl.ds(toff, TAIL)], si0.at[pl.ds(0, TAIL)])
    pltpu.sync_copy(dst_hbm.at[pl.ds(toff, TAIL)], di0.at[pl.ds(0, TAIL)])
    ct_s = pltpu.make_async_copy(h_sh.at[si0.at[pl.ds(0, TAIL)]],
                                 rs0.at[pl.ds(0, TAIL)], sem_s0)
    ct_d = pltpu.make_async_copy(h_sh.at[di0.at[pl.ds(0, TAIL)]],
                                 rd0.at[pl.ds(0, TAIL)], sem_d0)
    ct_s.start()
    ct_d.start()
    ct_s.wait()
    ct_d.wait()
    ob[pl.ds(0, TAIL)] = dots16(rs0, rd0, 0)
    pltpu.sync_copy(ob.at[pl.ds(0, TAIL)], out_hbm.at[pl.ds(toff, TAIL)])


@jax.jit
def _scores(src, dst, h):
    mesh = plsc.VectorSubcoreMesh(core_axis_name="c", subcore_axis_name="s")
    return pl.kernel(
        _body,
        out_type=jax.ShapeDtypeStruct((N_EDGES,), jnp.float32),
        mesh=mesh,
        scratch_types=[
            pltpu.VMEM_SHARED((N_NODES, D), jnp.float32),
            pltpu.VMEM((C,), jnp.int32),
            pltpu.VMEM((C,), jnp.int32),
            pltpu.VMEM((C,), jnp.int32),
            pltpu.VMEM((C,), jnp.int32),
            pltpu.VMEM((C, D), jnp.float32),
            pltpu.VMEM((C, D), jnp.float32),
            pltpu.VMEM((C, D), jnp.float32),
            pltpu.VMEM((C, D), jnp.float32),
            pltpu.VMEM((C,), jnp.float32),
            pltpu.SemaphoreType.DMA,
            pltpu.SemaphoreType.DMA,
            pltpu.SemaphoreType.DMA,
            pltpu.SemaphoreType.DMA,
        ],
    )(src, dst, h)


def kernel(edge_index, h):
    ei = edge_index.astype(jnp.int32)
    scores = _scores(ei[0], ei[1], h)
    return scores.reshape(N_EDGES, 1)
